# Initial kernel scaffold; baseline (speedup 1.0000x reference)
import jax
import jax.numpy as jnp
from jax.experimental import pallas as pl


def kernel(x, edge_index, batch, W1, b1, W2, b2):
    raise NotImplementedError("write your pallas kernel here")


# speedup vs baseline: 183.5613x; 183.5613x over previous
import functools

import jax
import jax.numpy as jnp
from jax import lax
from jax.experimental import pallas as pl
from jax.experimental.pallas import tpu as pltpu
from jax.experimental.pallas import tpu_sc as plsc

NC = 2
NS = 16
NW = NC * NS
LANES = 16
UNROLL = 4


def _sc_mesh():
  return plsc.VectorSubcoreMesh(core_axis_name="c", subcore_axis_name="s")


def _deg_kernel(nacc, ept, col1, zeros1, out, colv, acc):
  c = lax.axis_index("c")
  s = lax.axis_index("s")
  wid = c * NS + s
  pltpu.sync_copy(zeros1, acc)
  pltpu.sync_copy(col1.at[pl.ds(wid * ept, ept)], colv)
  ones16 = jnp.full((LANES,), 1.0, jnp.float32)

  @plsc.parallel_loop(0, ept // LANES, 1, unroll=UNROLL)
  def _(i):
    c16 = colv[pl.ds(i * LANES, LANES)]
    plsc.addupdate_scatter(acc, [c16], ones16)

  pltpu.sync_copy(acc, out.at[wid])


def _edge_kernel(nacc, ept, fdim, row1, col1, q, zerosq, out,
                 rowv, colv, qv, acc):
  c = lax.axis_index("c")
  s = lax.axis_index("s")
  wid = c * NS + s
  pltpu.sync_copy(zerosq, acc)
  pltpu.sync_copy(q, qv)
  pltpu.sync_copy(row1.at[pl.ds(wid * ept, ept)], rowv)
  pltpu.sync_copy(col1.at[pl.ds(wid * ept, ept)], colv)

  @plsc.parallel_loop(0, ept // LANES, 1, unroll=UNROLL)
  def _(i):
    base = i * LANES
    r16 = rowv[pl.ds(base, LANES)]
    c16 = colv[pl.ds(base, LANES)]
    for k in range(fdim):
      k16 = jnp.full((LANES,), k, jnp.int32)
      vals = plsc.load_gather(qv, [k16, r16])
      plsc.addupdate_scatter(acc, [k16, c16], vals)

  pltpu.sync_copy(acc, out.at[wid])


_SC_PARAMS = pltpu.CompilerParams(needs_layout_passes=False)


def _sc_deg(nacc, ept, col1, zeros1):
  return pl.kernel(
      functools.partial(_deg_kernel, nacc, ept),
      out_type=jax.ShapeDtypeStruct((NW, nacc), jnp.float32),
      mesh=_sc_mesh(),
      compiler_params=_SC_PARAMS,
      scratch_types=[
          pltpu.VMEM((ept,), jnp.int32),
          pltpu.VMEM((nacc,), jnp.float32),
      ],
  )(col1, zeros1)


def _sc_edge(nacc, ept, fdim, row1, col1, q, zerosq):
  return pl.kernel(
      functools.partial(_edge_kernel, nacc, ept, fdim),
      out_type=jax.ShapeDtypeStruct((NW, fdim, nacc), jnp.float32),
      mesh=_sc_mesh(),
      compiler_params=_SC_PARAMS,
      scratch_types=[
          pltpu.VMEM((ept,), jnp.int32),
          pltpu.VMEM((ept,), jnp.int32),
          pltpu.VMEM((fdim, nacc), jnp.float32),
          pltpu.VMEM((fdim, nacc), jnp.float32),
      ],
  )(row1, col1, q, zerosq)


def _prep_body(degp, xT, dinv_out, q1_out):
  deg = jnp.sum(degp[...], axis=0) + 1.0
  dinv = lax.rsqrt(deg)
  dinv_out[...] = dinv
  q1_out[...] = xT[...] * dinv


def _mid_body(s1p, q1, dinv, w1T, b1c, w2T, q2_out):
  s1 = jnp.sum(s1p[...], axis=0)
  p1 = dinv[...] * (s1 + q1[...])
  h = jnp.dot(w1T[...], p1, preferred_element_type=jnp.float32) + b1c[...]
  h = jnp.maximum(h, 0.0)
  t = jnp.dot(w2T[...], h, preferred_element_type=jnp.float32)
  q2_out[...] = t * dinv[...]


def _tail_body(g, s2p, q2, dinv, b2c, batchc, out):
  s2 = jnp.sum(s2p[...], axis=0)
  out2 = dinv[...] * (s2 + q2[...]) + b2c[...]
  nacc = out2.shape[1]
  gids = lax.broadcasted_iota(jnp.int32, (nacc, g), 1)
  ohT = (batchc[...] == gids).astype(jnp.float32)
  sums = jnp.dot(out2, ohT, preferred_element_type=jnp.float32)
  cnts = jnp.sum(ohT, axis=0, keepdims=True)
  pooled = sums / jnp.maximum(cnts, 1.0)
  m = jnp.max(pooled, axis=0, keepdims=True)
  z = pooled - m
  lse = jnp.log(jnp.sum(jnp.exp(z), axis=0, keepdims=True))
  out[...] = z - lse


def _tc_call(body, out_shape, *args):
  return pl.pallas_call(
      body,
      out_shape=jax.ShapeDtypeStruct(out_shape, jnp.float32),
  )(*args)


def _tc_prep(degp3, xT):
  return pl.pallas_call(
      _prep_body,
      out_shape=(jax.ShapeDtypeStruct((1, xT.shape[1]), jnp.float32),
                 jax.ShapeDtypeStruct(xT.shape, jnp.float32)),
  )(degp3, xT)


def kernel(x, edge_index, batch, W1, b1, W2, b2):
  n, fin = x.shape
  e = edge_index.shape[1]
  g = 64
  hid = W1.shape[1]
  fout = W2.shape[1]

  grp_e = LANES * UNROLL
  ept = -(-e // (NW * grp_e)) * grp_e
  e_pad = NW * ept
  nacc = -(-(n + 1) // 128) * 128

  row = edge_index[0]
  col = edge_index[1]
  pad = e_pad - e
  if pad:
    row = jnp.concatenate([row, jnp.zeros((pad,), jnp.int32)])
    col = jnp.concatenate([col, jnp.full((pad,), n, jnp.int32)])

  xT = jnp.zeros((fin, nacc), jnp.float32).at[:, :n].set(x.T)
  w1T = W1.T
  b1c = b1.reshape(hid, 1)
  w2T = W2.T
  b2c = b2.reshape(fout, 1)
  batchc = jnp.full((nacc, 1), g, jnp.int32).at[:n, 0].set(batch)
  zeros1 = jnp.zeros((nacc,), jnp.float32)
  zeros2 = jnp.zeros((fin, nacc), jnp.float32)
  zeros3 = jnp.zeros((fout, nacc), jnp.float32)

  degp = _sc_deg(nacc, ept, col, zeros1)
  dinv, q1 = _tc_prep(degp.reshape(NW, 1, nacc), xT)
  s1p = _sc_edge(nacc, ept, fin, row, col, q1, zeros2)
  q2 = _tc_call(_mid_body, (fout, nacc), s1p, q1, dinv, w1T, b1c, w2T)
  s2p = _sc_edge(nacc, ept, fout, row, col, q2, zeros3)
  out = _tc_call(functools.partial(_tail_body, g), (fout, g),
                 s2p, q2, dinv, b2c, batchc)
  return out.T
